# baseline (speedup 1.0000x reference)
import functools

import jax
import jax.numpy as jnp
from jax import lax
from jax.experimental import pallas as pl
from jax.experimental.pallas import tpu as pltpu
from jax.experimental.pallas import tpu_sc as plsc


def _embedding_gather(idx_flat, emb):
    B = idx_flat.shape[0]
    V, D = emb.shape
    info = plsc.get_sparse_core_info()
    NC, NS = info.num_cores, info.num_subcores
    NW = NC * NS
    assert B % NW == 0
    b_per_w = B // NW
    CH = 64
    assert b_per_w % CH == 0

    mesh = plsc.VectorSubcoreMesh(core_axis_name="c", subcore_axis_name="s")

    @functools.partial(
        pl.kernel,
        mesh=mesh,
        out_type=jax.ShapeDtypeStruct((B, D), jnp.float32),
        scratch_types=[
            pltpu.VMEM((CH,), jnp.int32),
            pltpu.VMEM((CH, D), jnp.float32),
            pltpu.SemaphoreType.DMA,
        ],
    )
    def gather_k(table_hbm, idx_hbm, out_hbm, idx_v, rows_v, sem):
        wid = lax.axis_index("s") * NC + lax.axis_index("c")
        base = wid * b_per_w
        for j in range(b_per_w // CH):
            off = base + j * CH
            pltpu.sync_copy(idx_hbm.at[pl.ds(off, CH)], idx_v)
            pltpu.async_copy(table_hbm.at[idx_v], rows_v, sem).wait()
            pltpu.sync_copy(rows_v, out_hbm.at[pl.ds(off, CH)])

    return gather_k(emb, idx_flat)


def _lstm_layer(x, h0, c0, wih_t, whh_t, bias, tch):
    SEQ, BSZ, D = x.shape
    H = whh_t.shape[0]

    G = SEQ // tch

    def body(x_ref, wih_ref, whh_ref, b_ref, h0_ref, c0_ref, y_ref,
             h_s, c_s, gx_s):
        i = pl.program_id(0)

        @pl.when(i == 0)
        def _init():
            h_s[...] = h0_ref[...]
            c_s[...] = c0_ref[...]

        cur = lax.rem(i, 2) * (tch * BSZ)
        prv = lax.rem(i + 1, 2) * (tch * BSZ)

        def step(t, _):
            @pl.when(i < G)
            def _fill():
                xb = x_ref[t].astype(jnp.bfloat16)
                gx_s[pl.ds(cur + t * BSZ, BSZ), :] = (
                    jnp.dot(xb, wih_ref[...],
                            preferred_element_type=jnp.float32)
                    + b_ref[...]
                )

            @pl.when(i > 0)
            def _recur():
                h = h_s[...]
                gates = gx_s[pl.ds(prv + t * BSZ, BSZ), :] + jnp.dot(
                    h.astype(jnp.bfloat16), whh_ref[...],
                    preferred_element_type=jnp.float32)
                ig = jax.nn.sigmoid(gates[:, :H])
                fg = jax.nn.sigmoid(gates[:, H:2 * H])
                gg = jnp.tanh(gates[:, 2 * H:3 * H])
                og = jax.nn.sigmoid(gates[:, 3 * H:])
                c = fg * c_s[...] + ig * gg
                hn = og * jnp.tanh(c)
                c_s[...] = c
                h_s[...] = hn
                y_ref[pl.ds(t, 1)] = hn[None]

            return 0

        lax.fori_loop(0, tch, step, 0)

    return pl.pallas_call(
        body,
        grid=(G + 1,),
        in_specs=[
            pl.BlockSpec((tch, BSZ, D),
                         lambda i: (jnp.minimum(i, G - 1), 0, 0)),
            pl.BlockSpec((D, 4 * H), lambda i: (0, 0)),
            pl.BlockSpec((H, 4 * H), lambda i: (0, 0)),
            pl.BlockSpec((1, 4 * H), lambda i: (0, 0)),
            pl.BlockSpec((BSZ, H), lambda i: (0, 0)),
            pl.BlockSpec((BSZ, H), lambda i: (0, 0)),
        ],
        out_specs=pl.BlockSpec((tch, BSZ, H),
                               lambda i: (jnp.maximum(i - 1, 0), 0, 0)),
        out_shape=jax.ShapeDtypeStruct((SEQ, BSZ, H), jnp.float32),
        scratch_shapes=[
            pltpu.VMEM((BSZ, H), jnp.float32),
            pltpu.VMEM((BSZ, H), jnp.float32),
            pltpu.VMEM((2 * tch * BSZ, 4 * H), jnp.float32),
        ],
        compiler_params=pltpu.CompilerParams(
            dimension_semantics=("arbitrary",)),
    )(x, wih_t, whh_t, bias, h0, c0)


def kernel(inp, emb,
           W_ih_0, W_hh_0, b_ih_0, b_hh_0, h_0, c_0,
           W_ih_1, W_hh_1, b_ih_1, b_hh_1, h_1, c_1,
           W_ih_2, W_hh_2, b_ih_2, b_hh_2, h_2, c_2):
    SEQ, BSZ = inp.shape
    D = emb.shape[1]
    H = W_hh_0.shape[1]

    idx_flat = inp.reshape(SEQ * BSZ).astype(jnp.int32)
    x = _embedding_gather(idx_flat, emb).reshape(SEQ, BSZ, D)

    layers = [
        (W_ih_0, W_hh_0, b_ih_0, b_hh_0, h_0, c_0),
        (W_ih_1, W_hh_1, b_ih_1, b_hh_1, h_1, c_1),
        (W_ih_2, W_hh_2, b_ih_2, b_hh_2, h_2, c_2),
    ]
    for (Wih, Whh, bih, bhh, h0, c0) in layers:
        x = _lstm_layer(
            x, h0, c0,
            jnp.transpose(Wih).astype(jnp.bfloat16),
            jnp.transpose(Whh).astype(jnp.bfloat16),
            (bih + bhh).reshape(1, 4 * H),
            tch=8,
        )
    return x.reshape(SEQ * BSZ, H)


# speedup vs baseline: 1.4793x; 1.4793x over previous
import functools

import jax
import jax.numpy as jnp
from jax import lax
from jax.experimental import pallas as pl
from jax.experimental.pallas import tpu as pltpu
from jax.experimental.pallas import tpu_sc as plsc


def _embedding_gather(idx_flat, emb):
    B = idx_flat.shape[0]
    V, D = emb.shape
    info = plsc.get_sparse_core_info()
    NC, NS = info.num_cores, info.num_subcores
    NW = NC * NS
    assert B % NW == 0
    b_per_w = B // NW
    CH = 64
    assert b_per_w % CH == 0

    mesh = plsc.VectorSubcoreMesh(core_axis_name="c", subcore_axis_name="s")

    @functools.partial(
        pl.kernel,
        mesh=mesh,
        out_type=jax.ShapeDtypeStruct((B, D), jnp.float32),
        scratch_types=[
            pltpu.VMEM((CH,), jnp.int32),
            pltpu.VMEM((CH, D), jnp.float32),
            pltpu.SemaphoreType.DMA,
        ],
    )
    def gather_k(table_hbm, idx_hbm, out_hbm, idx_v, rows_v, sem):
        wid = lax.axis_index("s") * NC + lax.axis_index("c")
        base = wid * b_per_w
        for j in range(b_per_w // CH):
            off = base + j * CH
            pltpu.sync_copy(idx_hbm.at[pl.ds(off, CH)], idx_v)
            pltpu.async_copy(table_hbm.at[idx_v], rows_v, sem).wait()
            pltpu.sync_copy(rows_v, out_hbm.at[pl.ds(off, CH)])

    return gather_k(emb, idx_flat)


def _lstm_layer(x, h0, c0, wih_t, whh_t, bias, tch):
    SEQ, BSZ, D = x.shape
    H = whh_t.shape[0]

    def body(x_ref, wih_ref, whh_ref, b_ref, h0_ref, c0_ref, y_ref,
             h_s, c_s, gx_s):
        i = pl.program_id(0)

        @pl.when(i == 0)
        def _init():
            h_s[...] = h0_ref[...]
            c_s[...] = c0_ref[...]

        xb = x_ref[...].reshape(tch * BSZ, D).astype(jnp.bfloat16)
        gx_s[...] = (
            jnp.dot(xb, wih_ref[...], preferred_element_type=jnp.float32)
            + b_ref[...]
        )

        for t in range(tch):
            h = h_s[...]
            gates = gx_s[t * BSZ:(t + 1) * BSZ, :] + jnp.dot(
                h.astype(jnp.bfloat16), whh_ref[...],
                preferred_element_type=jnp.float32)
            ig = jax.nn.sigmoid(gates[:, :H])
            fg = jax.nn.sigmoid(gates[:, H:2 * H])
            gg = jnp.tanh(gates[:, 2 * H:3 * H])
            og = jax.nn.sigmoid(gates[:, 3 * H:])
            c = fg * c_s[...] + ig * gg
            h = og * jnp.tanh(c)
            c_s[...] = c
            h_s[...] = h
            y_ref[t] = h

    return pl.pallas_call(
        body,
        grid=(SEQ // tch,),
        in_specs=[
            pl.BlockSpec((tch, BSZ, D), lambda i: (i, 0, 0)),
            pl.BlockSpec((D, 4 * H), lambda i: (0, 0)),
            pl.BlockSpec((H, 4 * H), lambda i: (0, 0)),
            pl.BlockSpec((1, 4 * H), lambda i: (0, 0)),
            pl.BlockSpec((BSZ, H), lambda i: (0, 0)),
            pl.BlockSpec((BSZ, H), lambda i: (0, 0)),
        ],
        out_specs=pl.BlockSpec((tch, BSZ, H), lambda i: (i, 0, 0)),
        out_shape=jax.ShapeDtypeStruct((SEQ, BSZ, H), jnp.float32),
        scratch_shapes=[
            pltpu.VMEM((BSZ, H), jnp.float32),
            pltpu.VMEM((BSZ, H), jnp.float32),
            pltpu.VMEM((tch * BSZ, 4 * H), jnp.float32),
        ],
        compiler_params=pltpu.CompilerParams(
            dimension_semantics=("arbitrary",)),
    )(x, wih_t, whh_t, bias, h0, c0)


def kernel(inp, emb,
           W_ih_0, W_hh_0, b_ih_0, b_hh_0, h_0, c_0,
           W_ih_1, W_hh_1, b_ih_1, b_hh_1, h_1, c_1,
           W_ih_2, W_hh_2, b_ih_2, b_hh_2, h_2, c_2):
    SEQ, BSZ = inp.shape
    D = emb.shape[1]
    H = W_hh_0.shape[1]

    idx_flat = inp.reshape(SEQ * BSZ).astype(jnp.int32)
    x = _embedding_gather(idx_flat, emb).reshape(SEQ, BSZ, D)

    layers = [
        (W_ih_0, W_hh_0, b_ih_0, b_hh_0, h_0, c_0),
        (W_ih_1, W_hh_1, b_ih_1, b_hh_1, h_1, c_1),
        (W_ih_2, W_hh_2, b_ih_2, b_hh_2, h_2, c_2),
    ]
    for (Wih, Whh, bih, bhh, h0, c0) in layers:
        x = _lstm_layer(
            x, h0, c0,
            jnp.transpose(Wih).astype(jnp.bfloat16),
            jnp.transpose(Whh).astype(jnp.bfloat16),
            (bih + bhh).reshape(1, 4 * H),
            tch=8,
        )
    return x.reshape(SEQ * BSZ, H)
